# Initial kernel scaffold; baseline (speedup 1.0000x reference)
import jax
import jax.numpy as jnp
from jax.experimental import pallas as pl


def kernel(logits, top_ps, top_ks):
    raise NotImplementedError("write your pallas kernel here")


# speedup vs baseline: 16.4041x; 16.4041x over previous
import jax
import jax.numpy as jnp
from jax.experimental import pallas as pl
from jax.experimental.pallas import tpu as pltpu


def _sampler_kernel(l_ref, p_ref, k_ref, out_ref):
    l = l_ref[...]
    R, V = l.shape
    p = p_ref[...]
    k = k_ref[...]

    m_row = jnp.max(l, axis=1, keepdims=True)
    e = jnp.exp(l - m_row)
    z = jnp.sum(e, axis=1, keepdims=True)
    pz = p * z

    b = jax.lax.bitcast_convert_type(l, jnp.int32)
    key = jnp.where(b >= 0, b, b ^ jnp.int32(0x7FFFFFFF))

    lo = jnp.min(key, axis=1, keepdims=True)
    hi = jnp.max(key, axis=1, keepdims=True)

    def val_body(_, carry):
        lo, hi = carry
        mid = (lo >> 1) + (hi >> 1) + (lo & hi & 1)
        gt = key > mid
        g = jnp.sum(gt.astype(jnp.int32), axis=1, keepdims=True)
        s = jnp.sum(jnp.where(gt, e, 0.0), axis=1, keepdims=True)
        keep = (g < k) & (s <= pz)
        return jnp.where(keep, lo, mid + 1), jnp.where(keep, mid, hi)

    lo, _ = jax.lax.fori_loop(0, 32, val_body, (lo, hi))
    vstar = lo

    gt = key > vstar
    eq = key == vstar
    g = jnp.sum(gt.astype(jnp.int32), axis=1, keepdims=True)
    s = jnp.sum(jnp.where(gt, e, 0.0), axis=1, keepdims=True)
    m_cnt = jnp.sum(eq.astype(jnp.int32), axis=1, keepdims=True)

    bstar = jnp.where(vstar >= 0, vstar, vstar ^ jnp.int32(0x7FFFFFFF))
    estar = jnp.exp(jax.lax.bitcast_convert_type(bstar, jnp.float32) - m_row)

    m_f = m_cnt.astype(jnp.float32)
    cnt_f = jnp.floor((pz - s) / estar) + 1.0
    np_in = jnp.minimum(jnp.where(estar > 0, cnt_f, m_f), m_f)
    np_in = jnp.maximum(np_in, 1.0).astype(jnp.int32)
    n = jnp.minimum(k, g + np_in)
    r = n - g

    iota = jax.lax.broadcasted_iota(jnp.int32, (R, V), 1)
    ilo = jnp.zeros_like(r)
    ihi = jnp.full_like(r, V)

    def idx_body(_, carry):
        ilo, ihi = carry
        mid = (ilo + ihi) >> 1
        c = jnp.sum((eq & (iota < mid)).astype(jnp.int32), axis=1, keepdims=True)
        ok = c >= r
        return jnp.where(ok, ilo, mid + 1), jnp.where(ok, mid, ihi)

    ilo, _ = jax.lax.fori_loop(0, 17, idx_body, (ilo, ihi))

    kept = gt | (eq & (iota < ilo))
    ml = jnp.where(kept, l, -1e9)
    m2 = jnp.max(ml, axis=1, keepdims=True)
    e2 = jnp.exp(ml - m2)
    z2 = jnp.sum(e2, axis=1, keepdims=True)
    out_ref[...] = e2 / z2


def kernel(logits, top_ps, top_ks):
    B, V = logits.shape
    rows = 8 if B % 8 == 0 else 1
    p2 = top_ps.astype(jnp.float32).reshape(B, 1)
    k2 = top_ks.astype(jnp.int32).reshape(B, 1)
    return pl.pallas_call(
        _sampler_kernel,
        grid=(B // rows,),
        in_specs=[
            pl.BlockSpec((rows, V), lambda i: (i, 0)),
            pl.BlockSpec((rows, 1), lambda i: (i, 0)),
            pl.BlockSpec((rows, 1), lambda i: (i, 0)),
        ],
        out_specs=pl.BlockSpec((rows, V), lambda i: (i, 0)),
        out_shape=jax.ShapeDtypeStruct((B, V), jnp.float32),
        compiler_params=pltpu.CompilerParams(
            dimension_semantics=("arbitrary",),
        ),
    )(logits, p2, k2)
